# baseline (speedup 1.0000x reference)
import functools

import jax
import jax.numpy as jnp
from jax import lax
from jax.experimental import pallas as pl
from jax.experimental.pallas import tpu as pltpu
from jax.experimental.pallas import tpu_sc as plsc

_D_IN = 128
_H1 = 512
_H2 = 256
_D_OUT = 100
_B, _L = 4096, 50
_N_TOK = _B * _L

_NC, _NS = 2, 16
_NW = _NC * _NS
_CHUNK = 128
_TB = 8192

_SLICES = (8192, 32768, 65536, 98304)
assert sum(_SLICES) == _N_TOK


def _make_sc_gather(sl_tok):
    per_w = sl_tok // _NW
    nchunk = per_w // _CHUNK
    mesh = plsc.VectorSubcoreMesh(core_axis_name="c", subcore_axis_name="s")

    @functools.partial(
        pl.kernel,
        mesh=mesh,
        out_type=jax.ShapeDtypeStruct((sl_tok, _D_IN), jnp.float32),
        scratch_types=[
            pltpu.VMEM((nchunk, _CHUNK), jnp.int32),
            pltpu.VMEM((2, _CHUNK, _D_IN), jnp.float32),
            pltpu.SemaphoreType.DMA,
            pltpu.SemaphoreType.DMA,
        ],
    )
    def gather_k(emb_hbm, tok_hbm, out_hbm, idx_v, rows_v, gsem, wsem):
        wid = lax.axis_index("s") * _NC + lax.axis_index("c")
        base = wid * per_w
        pltpu.sync_copy(tok_hbm.at[wid], idx_v)

        def issue_gather(j, slot):
            pltpu.async_copy(emb_hbm.at[idx_v.at[j]], rows_v.at[slot], gsem)

        def issue_write(j, slot):
            pltpu.async_copy(
                rows_v.at[slot], out_hbm.at[pl.ds(base + j * _CHUNK, _CHUNK)], wsem
            )

        def wait_gather(slot):
            pltpu.make_async_copy(
                emb_hbm.at[idx_v.at[0]], rows_v.at[slot], gsem
            ).wait()

        def wait_write(slot):
            pltpu.make_async_copy(
                rows_v.at[slot], out_hbm.at[pl.ds(0, _CHUNK)], wsem
            ).wait()

        issue_gather(0, 0)

        def pair(g, _):
            j0 = 2 * g
            @pl.when(g >= 1)
            def _():
                wait_write(1)

            issue_gather(j0 + 1, 1)
            wait_gather(0)
            issue_write(j0, 0)
            wait_write(0)

            @pl.when(g < nchunk // 2 - 1)
            def _():
                issue_gather(j0 + 2, 0)

            wait_gather(1)
            issue_write(j0 + 1, 1)
            return 0

        lax.fori_loop(0, nchunk // 2, pair, 0)
        wait_write(1)

    return gather_k


_sc_gathers = {n: _make_sc_gather(n) for n in sorted(set(_SLICES))}


def _mlp_body(x_ref, w1_ref, b1_ref, w2_ref, b2_ref, w3_ref, b3_ref, o_ref):
    x = x_ref[...].astype(jnp.bfloat16)
    h = jnp.dot(x, w1_ref[...].astype(jnp.bfloat16),
                preferred_element_type=jnp.float32) + b1_ref[...]
    h = jnp.maximum(h, 0.0).astype(jnp.bfloat16)
    h = jnp.dot(h, w2_ref[...].astype(jnp.bfloat16),
                preferred_element_type=jnp.float32) + b2_ref[...]
    h = jnp.maximum(h, 0.0).astype(jnp.bfloat16)
    o = jnp.dot(h, w3_ref[...].astype(jnp.bfloat16),
                preferred_element_type=jnp.float32) + b3_ref[...]
    o_ref[...] = o.T


def _mlp_body_acc(acc_ref, x_ref, w1_ref, b1_ref, w2_ref, b2_ref, w3_ref,
                  b3_ref, o_ref):
    del acc_ref
    _mlp_body(x_ref, w1_ref, b1_ref, w2_ref, b2_ref, w3_ref, b3_ref, o_ref)


def _mlp_slice(acc, x, W1, b1, W2, b2, W3, b3, row0):
    nsteps = x.shape[0] // _TB
    step0 = row0 // _TB
    rep = lambda i: (0, 0)
    in_specs = [
        pl.BlockSpec((_TB, _D_IN), lambda i: (i, 0)),
        pl.BlockSpec((_D_IN, _H1), rep),
        pl.BlockSpec((1, _H1), rep),
        pl.BlockSpec((_H1, _H2), rep),
        pl.BlockSpec((1, _H2), rep),
        pl.BlockSpec((_H2, _D_OUT), rep),
        pl.BlockSpec((1, _D_OUT), rep),
    ]
    args = (x, W1, b1.reshape(1, _H1), W2, b2.reshape(1, _H2), W3,
            b3.reshape(1, _D_OUT))
    body = _mlp_body
    aliases = {}
    if acc is not None:
        in_specs = [pl.BlockSpec(memory_space=pl.ANY)] + in_specs
        args = (acc,) + args
        body = _mlp_body_acc
        aliases = {0: 0}
    return pl.pallas_call(
        body,
        grid=(nsteps,),
        in_specs=in_specs,
        out_specs=pl.BlockSpec((_D_OUT, _TB), lambda i: (0, step0 + i)),
        out_shape=jax.ShapeDtypeStruct((_D_OUT, _N_TOK), jnp.float32),
        input_output_aliases=aliases,
        compiler_params=pltpu.CompilerParams(
            dimension_semantics=("arbitrary",),
        ),
    )(*args)


def kernel(token_ids, emb, W1, b1, W2, b2, W3, b3):
    tok = token_ids.reshape(-1).astype(jnp.int32)
    xs = []
    off = 0
    for n in _SLICES:
        t = tok[off:off + n].reshape(_NW, n // _NW // _CHUNK, _CHUNK)
        xs.append(_sc_gathers[n](emb, t))
        off += n
    acc = None
    off = 0
    for n, x in zip(_SLICES, xs):
        acc = _mlp_slice(acc, x, W1, b1, W2, b2, W3, b3, off)
        off += n
    return acc.T


# speedup vs baseline: 1.0190x; 1.0190x over previous
import functools

import jax
import jax.numpy as jnp
from jax import lax
from jax.experimental import pallas as pl
from jax.experimental.pallas import tpu as pltpu
from jax.experimental.pallas import tpu_sc as plsc

_D_IN = 128
_H1 = 512
_H2 = 256
_D_OUT = 100
_B, _L = 4096, 50
_N_TOK = _B * _L

_NC, _NS = 2, 16
_NW = _NC * _NS
_CHUNK = 128
_TB = 8192

_SLICES = (16384, 40960, 65536, 81920)
assert sum(_SLICES) == _N_TOK


def _make_sc_gather(sl_tok):
    per_w = sl_tok // _NW
    nchunk = per_w // _CHUNK
    mesh = plsc.VectorSubcoreMesh(core_axis_name="c", subcore_axis_name="s")

    @functools.partial(
        pl.kernel,
        mesh=mesh,
        out_type=jax.ShapeDtypeStruct((sl_tok, _D_IN), jnp.float32),
        scratch_types=[
            pltpu.VMEM((nchunk, _CHUNK), jnp.int32),
            pltpu.VMEM((2, _CHUNK, _D_IN), jnp.float32),
            pltpu.SemaphoreType.DMA,
            pltpu.SemaphoreType.DMA,
        ],
    )
    def gather_k(emb_hbm, tok_hbm, out_hbm, idx_v, rows_v, gsem, wsem):
        wid = lax.axis_index("s") * _NC + lax.axis_index("c")
        base = wid * per_w
        pltpu.sync_copy(tok_hbm.at[wid], idx_v)

        def issue_gather(j, slot):
            pltpu.async_copy(emb_hbm.at[idx_v.at[j]], rows_v.at[slot], gsem)

        def issue_write(j, slot):
            pltpu.async_copy(
                rows_v.at[slot], out_hbm.at[pl.ds(base + j * _CHUNK, _CHUNK)], wsem
            )

        def wait_gather(slot):
            pltpu.make_async_copy(
                emb_hbm.at[idx_v.at[0]], rows_v.at[slot], gsem
            ).wait()

        def wait_write(slot):
            pltpu.make_async_copy(
                rows_v.at[slot], out_hbm.at[pl.ds(0, _CHUNK)], wsem
            ).wait()

        issue_gather(0, 0)

        def pair(g, _):
            j0 = 2 * g
            @pl.when(g >= 1)
            def _():
                wait_write(1)

            issue_gather(j0 + 1, 1)
            wait_gather(0)
            issue_write(j0, 0)
            wait_write(0)

            @pl.when(g < nchunk // 2 - 1)
            def _():
                issue_gather(j0 + 2, 0)

            wait_gather(1)
            issue_write(j0 + 1, 1)
            return 0

        lax.fori_loop(0, nchunk // 2, pair, 0)
        wait_write(1)

    return gather_k


_sc_gathers = {n: _make_sc_gather(n) for n in sorted(set(_SLICES))}


def _mlp_body(x_ref, w1_ref, b1_ref, w2_ref, b2_ref, w3_ref, b3_ref, o_ref):
    x = x_ref[...].astype(jnp.bfloat16)
    h = jnp.dot(x, w1_ref[...].astype(jnp.bfloat16),
                preferred_element_type=jnp.float32) + b1_ref[...]
    h = jnp.maximum(h, 0.0).astype(jnp.bfloat16)
    h = jnp.dot(h, w2_ref[...].astype(jnp.bfloat16),
                preferred_element_type=jnp.float32) + b2_ref[...]
    h = jnp.maximum(h, 0.0).astype(jnp.bfloat16)
    o = jnp.dot(h, w3_ref[...].astype(jnp.bfloat16),
                preferred_element_type=jnp.float32) + b3_ref[...]
    o_ref[...] = o.T


def _mlp_body_acc(acc_ref, x_ref, w1_ref, b1_ref, w2_ref, b2_ref, w3_ref,
                  b3_ref, o_ref):
    del acc_ref
    _mlp_body(x_ref, w1_ref, b1_ref, w2_ref, b2_ref, w3_ref, b3_ref, o_ref)


def _mlp_slice(acc, x, W1, b1, W2, b2, W3, b3, row0):
    nsteps = x.shape[0] // _TB
    step0 = row0 // _TB
    rep = lambda i: (0, 0)
    in_specs = [
        pl.BlockSpec((_TB, _D_IN), lambda i: (i, 0)),
        pl.BlockSpec((_D_IN, _H1), rep),
        pl.BlockSpec((1, _H1), rep),
        pl.BlockSpec((_H1, _H2), rep),
        pl.BlockSpec((1, _H2), rep),
        pl.BlockSpec((_H2, _D_OUT), rep),
        pl.BlockSpec((1, _D_OUT), rep),
    ]
    args = (x, W1, b1.reshape(1, _H1), W2, b2.reshape(1, _H2), W3,
            b3.reshape(1, _D_OUT))
    body = _mlp_body
    aliases = {}
    if acc is not None:
        in_specs = [pl.BlockSpec(memory_space=pl.ANY)] + in_specs
        args = (acc,) + args
        body = _mlp_body_acc
        aliases = {0: 0}
    return pl.pallas_call(
        body,
        grid=(nsteps,),
        in_specs=in_specs,
        out_specs=pl.BlockSpec((_D_OUT, _TB), lambda i: (0, step0 + i)),
        out_shape=jax.ShapeDtypeStruct((_D_OUT, _N_TOK), jnp.float32),
        input_output_aliases=aliases,
        compiler_params=pltpu.CompilerParams(
            dimension_semantics=("arbitrary",),
        ),
    )(*args)


def kernel(token_ids, emb, W1, b1, W2, b2, W3, b3):
    tok = token_ids.reshape(-1).astype(jnp.int32)
    xs = []
    off = 0
    for n in _SLICES:
        t = tok[off:off + n].reshape(_NW, n // _NW // _CHUNK, _CHUNK)
        xs.append(_sc_gathers[n](emb, t))
        off += n
    acc = None
    off = 0
    for n, x in zip(_SLICES, xs):
        acc = _mlp_slice(acc, x, W1, b1, W2, b2, W3, b3, off)
        off += n
    return acc.T
